# baseline (speedup 1.0000x reference)
import functools

import jax
import jax.numpy as jnp
from jax import lax
from jax.experimental import pallas as pl
from jax.experimental.pallas import tpu as pltpu
from jax.experimental.pallas import tpu_sc as plsc

_BATCH = 16384
_VIS = 512
_EMB = 8
_HID = 16
_NUSER = 1000000
_NCAT = 368

_NC = 2
_NS = 16
_NW = _NC * _NS
_BPW = _BATCH // _NW
_CH = 128
_ROWS = _NUSER // _EMB


def _sc_gather_body(u1d, i1d, c1d, uembT, iembT, ubias8, ibias8, catT, cb1d,
                    u_o, i_o, c_o, bu_o, bi_o, bc_o,
                    uraw, iraw, craw, ush, ish,
                    gu, gi, gbu, gbi, catv, cbv,
                    outu, outi, outc, outbu, outbi, outbc, sem):
    c = lax.axis_index("c")
    s = lax.axis_index("s")
    wid = s * _NC + c
    base = wid * _BPW
    pltpu.sync_copy(u1d.at[pl.ds(base, _BPW)], uraw)
    pltpu.sync_copy(i1d.at[pl.ds(base, _BPW)], iraw)
    pltpu.sync_copy(c1d.at[pl.ds(base, _BPW)], craw)
    pltpu.sync_copy(catT, catv)
    pltpu.sync_copy(cb1d, cbv)

    lane = jnp.arange(16, dtype=jnp.int32)

    def idx_body(t, carry):
        p = t * 16 + lane
        d = p >> 9
        b = p & 511
        off = d * _ROWS
        u = plsc.load_gather(uraw, [b])
        ush[pl.ds(t * 16, 16)] = (u >> 3) + off
        i = plsc.load_gather(iraw, [b])
        ish[pl.ds(t * 16, 16)] = (i >> 3) + off
        return carry
    lax.fori_loop(0, 256, idx_body, 0)

    copies = []
    for k in range(32):
        sl = pl.ds(k * _CH, _CH)
        copies.append(pltpu.async_copy(uembT.at[ush.at[sl]], gu.at[sl], sem))
        copies.append(pltpu.async_copy(iembT.at[ish.at[sl]], gi.at[sl], sem))
    for k in range(4):
        sl = pl.ds(k * _CH, _CH)
        copies.append(pltpu.async_copy(ubias8.at[ush.at[sl]], gbu.at[sl], sem))
        copies.append(pltpu.async_copy(ibias8.at[ish.at[sl]], gbi.at[sl], sem))

    def cat_body(t, carry):
        p = t * 16 + lane
        b = p & 511
        d = p >> 9
        cid = plsc.load_gather(craw, [b])
        outc[pl.ds(t * 16, 16)] = plsc.load_gather(catv, [d, cid])
        return carry
    lax.fori_loop(0, 256, cat_body, 0)

    def cbias_body(t, carry):
        e = t * 16 + lane
        cid = plsc.load_gather(craw, [e])
        outbc[pl.ds(t * 16, 16)] = plsc.load_gather(cbv, [cid])
        return carry
    lax.fori_loop(0, 32, cbias_body, 0)

    for cp in copies:
        cp.wait()

    def sel_body(t, carry):
        p = t * 16 + lane
        b = p & 511
        u = plsc.load_gather(uraw, [b])
        outu[pl.ds(t * 16, 16)] = plsc.load_gather(gu, [p, u & 7])
        i = plsc.load_gather(iraw, [b])
        outi[pl.ds(t * 16, 16)] = plsc.load_gather(gi, [p, i & 7])
        return carry
    lax.fori_loop(0, 256, sel_body, 0)

    def bias_body(t, carry):
        e = t * 16 + lane
        u = plsc.load_gather(uraw, [e])
        outbu[pl.ds(t * 16, 16)] = plsc.load_gather(gbu, [e, u & 7])
        i = plsc.load_gather(iraw, [e])
        outbi[pl.ds(t * 16, 16)] = plsc.load_gather(gbi, [e, i & 7])
        return carry
    lax.fori_loop(0, 32, bias_body, 0)

    for d in range(_EMB):
        dsl = pl.ds(d * 512, 512)
        osl = pl.ds(d * _BATCH + base, _BPW)
        pltpu.sync_copy(outu.at[dsl], u_o.at[osl])
        pltpu.sync_copy(outi.at[dsl], i_o.at[osl])
        pltpu.sync_copy(outc.at[dsl], c_o.at[osl])
    pltpu.sync_copy(outbu, bu_o.at[pl.ds(base, _BPW)])
    pltpu.sync_copy(outbi, bi_o.at[pl.ds(base, _BPW)])
    pltpu.sync_copy(outbc, bc_o.at[pl.ds(base, _BPW)])


_sc_gather = functools.partial(
    pl.kernel,
    mesh=plsc.VectorSubcoreMesh(core_axis_name="c", subcore_axis_name="s"),
    out_type=(
        jax.ShapeDtypeStruct((_BATCH * _EMB,), jnp.float32),
        jax.ShapeDtypeStruct((_BATCH * _EMB,), jnp.float32),
        jax.ShapeDtypeStruct((_BATCH * _EMB,), jnp.float32),
        jax.ShapeDtypeStruct((_BATCH,), jnp.float32),
        jax.ShapeDtypeStruct((_BATCH,), jnp.float32),
        jax.ShapeDtypeStruct((_BATCH,), jnp.float32),
    ),
    scratch_types=(
        [pltpu.VMEM((_BPW,), jnp.int32) for _ in range(3)]
        + [pltpu.VMEM((_BPW * _EMB,), jnp.int32) for _ in range(2)]
        + [pltpu.VMEM((_BPW * _EMB, _EMB), jnp.float32) for _ in range(2)]
        + [pltpu.VMEM((_BPW, _EMB), jnp.float32) for _ in range(2)]
        + [pltpu.VMEM((_EMB, _NCAT), jnp.float32),
           pltpu.VMEM((_NCAT,), jnp.float32)]
        + [pltpu.VMEM((_BPW * _EMB,), jnp.float32) for _ in range(3)]
        + [pltpu.VMEM((_BPW,), jnp.float32) for _ in range(3)]
        + [pltpu.SemaphoreType.DMA]),
    compiler_params=pltpu.CompilerParams(use_tc_tiling_on_sc=False,
                                         needs_layout_passes=False),
)(_sc_gather_body)


_BLK = 2048
_NBLK = _BATCH // _BLK


def _mlp_body(vis_ref, w1t_ref, b1_ref, w2t_ref, b2_ref, out_ref):
    h = jnp.maximum(
        lax.dot_general(vis_ref[...], w1t_ref[...],
                        (((1,), (1,)), ((), ())),
                        preferred_element_type=jnp.float32) + b1_ref[...], 0.0)
    v = lax.dot_general(h, w2t_ref[...], (((1,), (1,)), ((), ())),
                        preferred_element_type=jnp.float32) + b2_ref[...]
    out_ref[...] = v.T


def _combine_body(scal_ref, u_ref, i_ref, c_ref, v_ref,
                  bu_ref, bi_ref, bc_ref, out_ref):
    w = scal_ref[0, 0]
    vb = scal_ref[0, 1]
    mn = scal_ref[0, 2]
    i2 = (1.0 - w) * i_ref[...] + w * (v_ref[...] + c_ref[...])
    pred = jnp.sum(u_ref[...] * i2, axis=0)
    out_ref[0, 0, :] = (pred + bu_ref[0, 0, :] + bi_ref[0, 0, :]
                        + w * (vb + bc_ref[0, 0, :]) + mn)


def kernel(u_id, i_id, weight, visual_features, category_features,
           user_emb, user_bias, item_emb, item_bias,
           W1, b1, W2, b2, visual_bias, category_emb, category_bias, mean):
    u_id = u_id.astype(jnp.int32)
    i_id = i_id.astype(jnp.int32)
    cat = category_features.astype(jnp.int32)

    U1, I1, C1, bu, bi, bc = _sc_gather(
        u_id, i_id, cat,
        user_emb.T.reshape(_NUSER, _EMB),
        item_emb.T.reshape(_NUSER, _EMB),
        user_bias.reshape(_NUSER // _EMB, _EMB),
        item_bias.reshape(_NUSER // _EMB, _EMB),
        category_emb.T,
        category_bias.reshape(_NCAT))

    vt = pl.pallas_call(
        _mlp_body,
        grid=(_NBLK,),
        in_specs=[
            pl.BlockSpec((_BLK, _VIS), lambda i: (i, 0)),
            pl.BlockSpec((_HID, _VIS), lambda i: (0, 0)),
            pl.BlockSpec((1, _HID), lambda i: (0, 0)),
            pl.BlockSpec((_EMB, _HID), lambda i: (0, 0)),
            pl.BlockSpec((1, _EMB), lambda i: (0, 0)),
        ],
        out_specs=pl.BlockSpec((_EMB, _BLK), lambda i: (0, i)),
        out_shape=jax.ShapeDtypeStruct((_EMB, _BATCH), jnp.float32),
    )(visual_features, W1.T, b1.reshape(1, _HID), W2.T, b2.reshape(1, _EMB))

    scal = jnp.concatenate([weight, visual_bias, mean]).reshape(1, 3)

    out = pl.pallas_call(
        _combine_body,
        grid=(_NBLK,),
        in_specs=[
            pl.BlockSpec(memory_space=pltpu.SMEM),
            pl.BlockSpec((_EMB, _BLK), lambda i: (0, i)),
            pl.BlockSpec((_EMB, _BLK), lambda i: (0, i)),
            pl.BlockSpec((_EMB, _BLK), lambda i: (0, i)),
            pl.BlockSpec((_EMB, _BLK), lambda i: (0, i)),
            pl.BlockSpec((1, 1, _BLK), lambda i: (i, 0, 0)),
            pl.BlockSpec((1, 1, _BLK), lambda i: (i, 0, 0)),
            pl.BlockSpec((1, 1, _BLK), lambda i: (i, 0, 0)),
        ],
        out_specs=pl.BlockSpec((1, 1, _BLK), lambda i: (i, 0, 0)),
        out_shape=jax.ShapeDtypeStruct((_NBLK, 1, _BLK), jnp.float32),
    )(scal, U1.reshape(_EMB, _BATCH), I1.reshape(_EMB, _BATCH),
      C1.reshape(_EMB, _BATCH), vt,
      bu.reshape(_NBLK, 1, _BLK), bi.reshape(_NBLK, 1, _BLK),
      bc.reshape(_NBLK, 1, _BLK))

    return out.reshape(_BATCH)


# speedup vs baseline: 5.7217x; 5.7217x over previous
import functools

import jax
import jax.numpy as jnp
from jax import lax
from jax.experimental import pallas as pl
from jax.experimental.pallas import tpu as pltpu
from jax.experimental.pallas import tpu_sc as plsc

_BATCH = 16384
_VIS = 512
_EMB = 8
_HID = 16
_NUSER = 1000000
_NCAT = 368

_NC = 2
_NS = 16
_NW = _NC * _NS
_BPW = _BATCH // _NW
_CH = 128
_TAIL = (_NUSER // _CH) * _CH

_CPW = 16384
_CPG = (_NUSER + _CPW - 1) // _CPW
_CPR = _CPW // _CH * _EMB


def _copy_body(in_ref, out_ref):
    for t in range(_CPW // _CH):
        out_ref[pl.ds(_EMB * t, _EMB), :] = in_ref[:, pl.ds(_CH * t, _CH)]


def _tile_stream(tabT):
    return pl.pallas_call(
        _copy_body,
        grid=(_CPG,),
        in_specs=[pl.BlockSpec((_EMB, _CPW), lambda i: (0, i))],
        out_specs=pl.BlockSpec((_CPR, _CH), lambda i: (i, 0)),
        out_shape=jax.ShapeDtypeStruct((_NUSER * _EMB // _CH, _CH),
                                       jnp.float32),
    )(tabT)


def _sc_gather_body(u1d, i1d, c1d, ulin, ilin, ub8, ib8, catf, cbf,
                    tailu, taili,
                    u_o, i_o, c_o, bu_o, bi_o, bc_o,
                    uraw, iraw, craw, ush, ish, ubsh, ibsh,
                    gu, gi, gbu, gbi, catv, cbv, tailuv, tailiv,
                    outu, outi, outc, outbu, outbi, outbc, sem):
    c = lax.axis_index("c")
    s = lax.axis_index("s")
    wid = s * _NC + c
    base = wid * _BPW
    pltpu.sync_copy(u1d.at[pl.ds(base, _BPW)], uraw)
    pltpu.sync_copy(i1d.at[pl.ds(base, _BPW)], iraw)
    pltpu.sync_copy(c1d.at[pl.ds(base, _BPW)], craw)
    pltpu.sync_copy(catf, catv)
    pltpu.sync_copy(cbf, cbv)
    pltpu.sync_copy(tailu, tailuv)
    pltpu.sync_copy(taili, tailiv)

    lane = jnp.arange(16, dtype=jnp.int32)

    def idx_body(t, carry):
        p = t * 16 + lane
        d16 = (p >> 9) * 16
        b = p & 511
        u = jnp.minimum(plsc.load_gather(uraw, [b]), _TAIL - 1)
        ush[pl.ds(t * 16, 16)] = (u >> 7) * 128 + d16 + ((u >> 3) & 15)
        i = jnp.minimum(plsc.load_gather(iraw, [b]), _TAIL - 1)
        ish[pl.ds(t * 16, 16)] = (i >> 7) * 128 + d16 + ((i >> 3) & 15)
        return carry
    lax.fori_loop(0, 256, idx_body, 0)

    def bidx_body(t, carry):
        e = t * 16 + lane
        ubsh[pl.ds(t * 16, 16)] = plsc.load_gather(uraw, [e]) >> 3
        ibsh[pl.ds(t * 16, 16)] = plsc.load_gather(iraw, [e]) >> 3
        return carry
    lax.fori_loop(0, 32, bidx_body, 0)

    copies = []
    for k in range(32):
        sl = pl.ds(k * _CH, _CH)
        copies.append(pltpu.async_copy(ulin.at[ush.at[sl]], gu.at[sl], sem))
        copies.append(pltpu.async_copy(ilin.at[ish.at[sl]], gi.at[sl], sem))
    for k in range(4):
        sl = pl.ds(k * _CH, _CH)
        copies.append(pltpu.async_copy(ub8.at[ubsh.at[sl]], gbu.at[sl], sem))
        copies.append(pltpu.async_copy(ib8.at[ibsh.at[sl]], gbi.at[sl], sem))

    def cat_body(t, carry):
        p = t * 16 + lane
        b = p & 511
        d = p >> 9
        cid = plsc.load_gather(craw, [b])
        outc[pl.ds(t * 16, 16)] = plsc.load_gather(catv, [d * _NCAT + cid])
        return carry
    lax.fori_loop(0, 256, cat_body, 0)

    def cbias_body(t, carry):
        e = t * 16 + lane
        cid = plsc.load_gather(craw, [e])
        outbc[pl.ds(t * 16, 16)] = plsc.load_gather(cbv, [cid])
        return carry
    lax.fori_loop(0, 32, cbias_body, 0)

    for cp in copies:
        cp.wait()

    def sel_body(t, carry):
        p = t * 16 + lane
        b = p & 511
        d = p >> 9
        u = plsc.load_gather(uraw, [b])
        mu = plsc.load_gather(gu, [p, u & 7])
        tu = plsc.load_gather(tailuv, [jnp.maximum(u - _TAIL, 0), d])
        outu[pl.ds(t * 16, 16)] = jnp.where(u >= _TAIL, tu, mu)
        i = plsc.load_gather(iraw, [b])
        mi = plsc.load_gather(gi, [p, i & 7])
        ti = plsc.load_gather(tailiv, [jnp.maximum(i - _TAIL, 0), d])
        outi[pl.ds(t * 16, 16)] = jnp.where(i >= _TAIL, ti, mi)
        return carry
    lax.fori_loop(0, 256, sel_body, 0)

    def bias_body(t, carry):
        e = t * 16 + lane
        u = plsc.load_gather(uraw, [e])
        outbu[pl.ds(t * 16, 16)] = plsc.load_gather(gbu, [e, u & 7])
        i = plsc.load_gather(iraw, [e])
        outbi[pl.ds(t * 16, 16)] = plsc.load_gather(gbi, [e, i & 7])
        return carry
    lax.fori_loop(0, 32, bias_body, 0)

    for d in range(_EMB):
        dsl = pl.ds(d * 512, 512)
        osl = pl.ds(d * _BATCH + base, _BPW)
        pltpu.sync_copy(outu.at[dsl], u_o.at[osl])
        pltpu.sync_copy(outi.at[dsl], i_o.at[osl])
        pltpu.sync_copy(outc.at[dsl], c_o.at[osl])
    pltpu.sync_copy(outbu, bu_o.at[pl.ds(base, _BPW)])
    pltpu.sync_copy(outbi, bi_o.at[pl.ds(base, _BPW)])
    pltpu.sync_copy(outbc, bc_o.at[pl.ds(base, _BPW)])


_sc_gather = functools.partial(
    pl.kernel,
    mesh=plsc.VectorSubcoreMesh(core_axis_name="c", subcore_axis_name="s"),
    out_type=(
        jax.ShapeDtypeStruct((_BATCH * _EMB,), jnp.float32),
        jax.ShapeDtypeStruct((_BATCH * _EMB,), jnp.float32),
        jax.ShapeDtypeStruct((_BATCH * _EMB,), jnp.float32),
        jax.ShapeDtypeStruct((_BATCH,), jnp.float32),
        jax.ShapeDtypeStruct((_BATCH,), jnp.float32),
        jax.ShapeDtypeStruct((_BATCH,), jnp.float32),
    ),
    scratch_types=(
        [pltpu.VMEM((_BPW,), jnp.int32) for _ in range(3)]
        + [pltpu.VMEM((_BPW * _EMB,), jnp.int32) for _ in range(2)]
        + [pltpu.VMEM((_BPW,), jnp.int32) for _ in range(2)]
        + [pltpu.VMEM((_BPW * _EMB, _EMB), jnp.float32) for _ in range(2)]
        + [pltpu.VMEM((_BPW, _EMB), jnp.float32) for _ in range(2)]
        + [pltpu.VMEM((_EMB * _NCAT,), jnp.float32),
           pltpu.VMEM((_NCAT,), jnp.float32)]
        + [pltpu.VMEM((_NUSER - _TAIL, _EMB), jnp.float32) for _ in range(2)]
        + [pltpu.VMEM((_BPW * _EMB,), jnp.float32) for _ in range(3)]
        + [pltpu.VMEM((_BPW,), jnp.float32) for _ in range(3)]
        + [pltpu.SemaphoreType.DMA]),
    compiler_params=pltpu.CompilerParams(use_tc_tiling_on_sc=False,
                                         needs_layout_passes=False),
)(_sc_gather_body)


_BLK = 2048
_NBLK = _BATCH // _BLK


def _mlp_body(vis_ref, w1t_ref, b1_ref, w2t_ref, b2_ref, out_ref):
    h = jnp.maximum(
        lax.dot_general(vis_ref[...], w1t_ref[...],
                        (((1,), (1,)), ((), ())),
                        preferred_element_type=jnp.float32) + b1_ref[...], 0.0)
    v = lax.dot_general(h, w2t_ref[...], (((1,), (1,)), ((), ())),
                        preferred_element_type=jnp.float32) + b2_ref[...]
    out_ref[...] = v.T


def _combine_body(scal_ref, u_ref, i_ref, c_ref, v_ref,
                  bu_ref, bi_ref, bc_ref, out_ref):
    w = scal_ref[0, 0]
    vb = scal_ref[0, 1]
    mn = scal_ref[0, 2]
    i2 = (1.0 - w) * i_ref[...] + w * (v_ref[...] + c_ref[...])
    pred = jnp.sum(u_ref[...] * i2, axis=0)
    out_ref[0, 0, :] = (pred + bu_ref[0, 0, :] + bi_ref[0, 0, :]
                        + w * (vb + bc_ref[0, 0, :]) + mn)


def kernel(u_id, i_id, weight, visual_features, category_features,
           user_emb, user_bias, item_emb, item_bias,
           W1, b1, W2, b2, visual_bias, category_emb, category_bias, mean):
    u_id = u_id.astype(jnp.int32)
    i_id = i_id.astype(jnp.int32)
    cat = category_features.astype(jnp.int32)

    ulin = _tile_stream(user_emb.T).reshape(_NUSER, _EMB)
    ilin = _tile_stream(item_emb.T).reshape(_NUSER, _EMB)

    U1, I1, C1, bu, bi, bc = _sc_gather(
        u_id, i_id, cat, ulin, ilin,
        user_bias.T.reshape(_NUSER // _EMB, _EMB),
        item_bias.T.reshape(_NUSER // _EMB, _EMB),
        category_emb.T.reshape(_EMB * _NCAT),
        category_bias.T.reshape(_NCAT),
        user_emb[_TAIL:], item_emb[_TAIL:])

    vt = pl.pallas_call(
        _mlp_body,
        grid=(_NBLK,),
        in_specs=[
            pl.BlockSpec((_BLK, _VIS), lambda i: (i, 0)),
            pl.BlockSpec((_HID, _VIS), lambda i: (0, 0)),
            pl.BlockSpec((1, _HID), lambda i: (0, 0)),
            pl.BlockSpec((_EMB, _HID), lambda i: (0, 0)),
            pl.BlockSpec((1, _EMB), lambda i: (0, 0)),
        ],
        out_specs=pl.BlockSpec((_EMB, _BLK), lambda i: (0, i)),
        out_shape=jax.ShapeDtypeStruct((_EMB, _BATCH), jnp.float32),
    )(visual_features, W1.T, b1.reshape(1, _HID), W2.T, b2.reshape(1, _EMB))

    scal = jnp.concatenate([weight, visual_bias, mean]).reshape(1, 3)

    out = pl.pallas_call(
        _combine_body,
        grid=(_NBLK,),
        in_specs=[
            pl.BlockSpec(memory_space=pltpu.SMEM),
            pl.BlockSpec((_EMB, _BLK), lambda i: (0, i)),
            pl.BlockSpec((_EMB, _BLK), lambda i: (0, i)),
            pl.BlockSpec((_EMB, _BLK), lambda i: (0, i)),
            pl.BlockSpec((_EMB, _BLK), lambda i: (0, i)),
            pl.BlockSpec((1, 1, _BLK), lambda i: (i, 0, 0)),
            pl.BlockSpec((1, 1, _BLK), lambda i: (i, 0, 0)),
            pl.BlockSpec((1, 1, _BLK), lambda i: (i, 0, 0)),
        ],
        out_specs=pl.BlockSpec((1, 1, _BLK), lambda i: (i, 0, 0)),
        out_shape=jax.ShapeDtypeStruct((_NBLK, 1, _BLK), jnp.float32),
    )(scal, U1.reshape(_EMB, _BATCH), I1.reshape(_EMB, _BATCH),
      C1.reshape(_EMB, _BATCH), vt,
      bu.reshape(_NBLK, 1, _BLK), bi.reshape(_NBLK, 1, _BLK),
      bc.reshape(_NBLK, 1, _BLK))

    return out.reshape(_BATCH)


# speedup vs baseline: 5.8962x; 1.0305x over previous
import functools

import jax
import jax.numpy as jnp
from jax import lax
from jax.experimental import pallas as pl
from jax.experimental.pallas import tpu as pltpu
from jax.experimental.pallas import tpu_sc as plsc

_BATCH = 16384
_VIS = 512
_EMB = 8
_HID = 16
_NUSER = 1000000
_NCAT = 368

_NC = 2
_NS = 16
_NW = _NC * _NS
_BPW = _BATCH // _NW
_CH = 128
_TAIL = (_NUSER // _CH) * _CH

_CPW = 16384
_CPG = (_NUSER + _CPW - 1) // _CPW
_CPR = _CPW // _CH * _EMB


def _copy_body(in_ref, out_ref):
    for t in range(_CPW // _CH):
        out_ref[pl.ds(_EMB * t, _EMB), :] = in_ref[:, pl.ds(_CH * t, _CH)]


def _tile_stream(tabT):
    return pl.pallas_call(
        _copy_body,
        grid=(_CPG,),
        in_specs=[pl.BlockSpec((_EMB, _CPW), lambda i: (0, i))],
        out_specs=pl.BlockSpec((_CPR, _CH), lambda i: (i, 0)),
        out_shape=jax.ShapeDtypeStruct((_NUSER * _EMB // _CH, _CH),
                                       jnp.float32),
    )(tabT)


def _sc_gather_body(u1d, i1d, c1d, ulin, ilin, ub8, ib8, catf, cbf,
                    tailu, taili,
                    u_o, i_o, c_o, bu_o, bi_o, bc_o,
                    uraw, iraw, craw, ush, ish, ubsh, ibsh,
                    gu, gi, gbu, gbi, catv, cbv, tailuv, tailiv,
                    outu, outi, outc, outbu, outbi, outbc, sem):
    c = lax.axis_index("c")
    s = lax.axis_index("s")
    wid = s * _NC + c
    base = wid * _BPW
    pltpu.sync_copy(u1d.at[pl.ds(base, _BPW)], uraw)
    pltpu.sync_copy(i1d.at[pl.ds(base, _BPW)], iraw)
    pltpu.sync_copy(c1d.at[pl.ds(base, _BPW)], craw)
    pltpu.sync_copy(catf, catv)
    pltpu.sync_copy(cbf, cbv)
    pltpu.sync_copy(tailu, tailuv)
    pltpu.sync_copy(taili, tailiv)

    lane = jnp.arange(16, dtype=jnp.int32)

    def idx_body(t, carry):
        sl16 = pl.ds(t * 16, 16)
        u = uraw[sl16]
        i = iraw[sl16]
        ubsh[sl16] = u >> 3
        ibsh[sl16] = i >> 3
        uc = jnp.minimum(u, _TAIL - 1)
        ic = jnp.minimum(i, _TAIL - 1)
        ubase = (uc >> 7) * 128 + ((uc >> 3) & 15)
        ibase = (ic >> 7) * 128 + ((ic >> 3) & 15)
        for d in range(_EMB):
            dsl = pl.ds(d * 512 + t * 16, 16)
            ush[dsl] = ubase + d * 16
            ish[dsl] = ibase + d * 16
        return carry
    lax.fori_loop(0, 32, idx_body, 0)

    copies = []
    for k in range(32):
        sl = pl.ds(k * _CH, _CH)
        copies.append(pltpu.async_copy(ulin.at[ush.at[sl]], gu.at[sl], sem))
        copies.append(pltpu.async_copy(ilin.at[ish.at[sl]], gi.at[sl], sem))
    for k in range(4):
        sl = pl.ds(k * _CH, _CH)
        copies.append(pltpu.async_copy(ub8.at[ubsh.at[sl]], gbu.at[sl], sem))
        copies.append(pltpu.async_copy(ib8.at[ibsh.at[sl]], gbi.at[sl], sem))

    def cat_body(t, carry):
        sl16 = pl.ds(t * 16, 16)
        cid = craw[sl16]
        outbc[sl16] = plsc.load_gather(cbv, [cid])
        for d in range(_EMB):
            outc[pl.ds(d * 512 + t * 16, 16)] = plsc.load_gather(
                catv, [d * _NCAT + cid])
        return carry
    lax.fori_loop(0, 32, cat_body, 0)

    for cp in copies:
        cp.wait()

    def sel_body(t, carry):
        sl16 = pl.ds(t * 16, 16)
        b = t * 16 + lane
        u = uraw[sl16]
        i = iraw[sl16]
        usel = u & 7
        isel = i & 7
        utail = u >= _TAIL
        itail = i >= _TAIL
        utidx = jnp.maximum(u - _TAIL, 0)
        itidx = jnp.maximum(i - _TAIL, 0)
        outbu[sl16] = plsc.load_gather(gbu, [b, usel])
        outbi[sl16] = plsc.load_gather(gbi, [b, isel])
        for d in range(_EMB):
            p = d * 512 + b
            dsl = pl.ds(d * 512 + t * 16, 16)
            dvec = jnp.full((16,), d, jnp.int32)
            mu = plsc.load_gather(gu, [p, usel])
            tu = plsc.load_gather(tailuv, [utidx, dvec])
            outu[dsl] = jnp.where(utail, tu, mu)
            mi = plsc.load_gather(gi, [p, isel])
            ti = plsc.load_gather(tailiv, [itidx, dvec])
            outi[dsl] = jnp.where(itail, ti, mi)
        return carry
    lax.fori_loop(0, 32, sel_body, 0)

    for d in range(_EMB):
        dsl = pl.ds(d * 512, 512)
        osl = pl.ds(d * _BATCH + base, _BPW)
        pltpu.sync_copy(outu.at[dsl], u_o.at[osl])
        pltpu.sync_copy(outi.at[dsl], i_o.at[osl])
        pltpu.sync_copy(outc.at[dsl], c_o.at[osl])
    pltpu.sync_copy(outbu, bu_o.at[pl.ds(base, _BPW)])
    pltpu.sync_copy(outbi, bi_o.at[pl.ds(base, _BPW)])
    pltpu.sync_copy(outbc, bc_o.at[pl.ds(base, _BPW)])


_sc_gather = functools.partial(
    pl.kernel,
    mesh=plsc.VectorSubcoreMesh(core_axis_name="c", subcore_axis_name="s"),
    out_type=(
        jax.ShapeDtypeStruct((_BATCH * _EMB,), jnp.float32),
        jax.ShapeDtypeStruct((_BATCH * _EMB,), jnp.float32),
        jax.ShapeDtypeStruct((_BATCH * _EMB,), jnp.float32),
        jax.ShapeDtypeStruct((_BATCH,), jnp.float32),
        jax.ShapeDtypeStruct((_BATCH,), jnp.float32),
        jax.ShapeDtypeStruct((_BATCH,), jnp.float32),
    ),
    scratch_types=(
        [pltpu.VMEM((_BPW,), jnp.int32) for _ in range(3)]
        + [pltpu.VMEM((_BPW * _EMB,), jnp.int32) for _ in range(2)]
        + [pltpu.VMEM((_BPW,), jnp.int32) for _ in range(2)]
        + [pltpu.VMEM((_BPW * _EMB, _EMB), jnp.float32) for _ in range(2)]
        + [pltpu.VMEM((_BPW, _EMB), jnp.float32) for _ in range(2)]
        + [pltpu.VMEM((_EMB * _NCAT,), jnp.float32),
           pltpu.VMEM((_NCAT,), jnp.float32)]
        + [pltpu.VMEM((_NUSER - _TAIL, _EMB), jnp.float32) for _ in range(2)]
        + [pltpu.VMEM((_BPW * _EMB,), jnp.float32) for _ in range(3)]
        + [pltpu.VMEM((_BPW,), jnp.float32) for _ in range(3)]
        + [pltpu.SemaphoreType.DMA]),
    compiler_params=pltpu.CompilerParams(use_tc_tiling_on_sc=False,
                                         needs_layout_passes=False),
)(_sc_gather_body)


_BLK = 2048
_NBLK = _BATCH // _BLK


def _mlp_body(vis_ref, w1t_ref, b1_ref, w2t_ref, b2_ref, out_ref):
    h = jnp.maximum(
        lax.dot_general(vis_ref[...], w1t_ref[...],
                        (((1,), (1,)), ((), ())),
                        preferred_element_type=jnp.float32) + b1_ref[...], 0.0)
    v = lax.dot_general(h, w2t_ref[...], (((1,), (1,)), ((), ())),
                        preferred_element_type=jnp.float32) + b2_ref[...]
    out_ref[...] = v.T


def _combine_body(scal_ref, u_ref, i_ref, c_ref, v_ref,
                  bu_ref, bi_ref, bc_ref, out_ref):
    w = scal_ref[0, 0]
    vb = scal_ref[0, 1]
    mn = scal_ref[0, 2]
    i2 = (1.0 - w) * i_ref[...] + w * (v_ref[...] + c_ref[...])
    pred = jnp.sum(u_ref[...] * i2, axis=0)
    out_ref[0, 0, :] = (pred + bu_ref[0, 0, :] + bi_ref[0, 0, :]
                        + w * (vb + bc_ref[0, 0, :]) + mn)


def kernel(u_id, i_id, weight, visual_features, category_features,
           user_emb, user_bias, item_emb, item_bias,
           W1, b1, W2, b2, visual_bias, category_emb, category_bias, mean):
    u_id = u_id.astype(jnp.int32)
    i_id = i_id.astype(jnp.int32)
    cat = category_features.astype(jnp.int32)

    ulin = _tile_stream(user_emb.T).reshape(_NUSER, _EMB)
    ilin = _tile_stream(item_emb.T).reshape(_NUSER, _EMB)

    U1, I1, C1, bu, bi, bc = _sc_gather(
        u_id, i_id, cat, ulin, ilin,
        user_bias.T.reshape(_NUSER // _EMB, _EMB),
        item_bias.T.reshape(_NUSER // _EMB, _EMB),
        category_emb.T.reshape(_EMB * _NCAT),
        category_bias.T.reshape(_NCAT),
        user_emb[_TAIL:], item_emb[_TAIL:])

    vt = pl.pallas_call(
        _mlp_body,
        grid=(_NBLK,),
        in_specs=[
            pl.BlockSpec((_BLK, _VIS), lambda i: (i, 0)),
            pl.BlockSpec((_HID, _VIS), lambda i: (0, 0)),
            pl.BlockSpec((1, _HID), lambda i: (0, 0)),
            pl.BlockSpec((_EMB, _HID), lambda i: (0, 0)),
            pl.BlockSpec((1, _EMB), lambda i: (0, 0)),
        ],
        out_specs=pl.BlockSpec((_EMB, _BLK), lambda i: (0, i)),
        out_shape=jax.ShapeDtypeStruct((_EMB, _BATCH), jnp.float32),
    )(visual_features, W1.T, b1.reshape(1, _HID), W2.T, b2.reshape(1, _EMB))

    scal = jnp.concatenate([weight, visual_bias, mean]).reshape(1, 3)

    out = pl.pallas_call(
        _combine_body,
        grid=(_NBLK,),
        in_specs=[
            pl.BlockSpec(memory_space=pltpu.SMEM),
            pl.BlockSpec((_EMB, _BLK), lambda i: (0, i)),
            pl.BlockSpec((_EMB, _BLK), lambda i: (0, i)),
            pl.BlockSpec((_EMB, _BLK), lambda i: (0, i)),
            pl.BlockSpec((_EMB, _BLK), lambda i: (0, i)),
            pl.BlockSpec((1, 1, _BLK), lambda i: (i, 0, 0)),
            pl.BlockSpec((1, 1, _BLK), lambda i: (i, 0, 0)),
            pl.BlockSpec((1, 1, _BLK), lambda i: (i, 0, 0)),
        ],
        out_specs=pl.BlockSpec((1, 1, _BLK), lambda i: (i, 0, 0)),
        out_shape=jax.ShapeDtypeStruct((_NBLK, 1, _BLK), jnp.float32),
    )(scal, U1.reshape(_EMB, _BATCH), I1.reshape(_EMB, _BATCH),
      C1.reshape(_EMB, _BATCH), vt,
      bu.reshape(_NBLK, 1, _BLK), bi.reshape(_NBLK, 1, _BLK),
      bc.reshape(_NBLK, 1, _BLK))

    return out.reshape(_BATCH)


# speedup vs baseline: 6.1122x; 1.0366x over previous
import functools

import jax
import jax.numpy as jnp
from jax import lax
from jax.experimental import pallas as pl
from jax.experimental.pallas import tpu as pltpu
from jax.experimental.pallas import tpu_sc as plsc

_BATCH = 16384
_VIS = 512
_EMB = 8
_HID = 16
_NUSER = 1000000
_NCAT = 368

_NC = 2
_NS = 16
_NW = _NC * _NS
_BPW = _BATCH // _NW
_CH = 128
_TAIL = (_NUSER // _CH) * _CH

_CPW = 16384
_CPG = (_NUSER + _CPW - 1) // _CPW
_CPR = _CPW // _CH * _EMB


def _copy_body(in_ref, out_ref):
    for t in range(_CPW // _CH):
        out_ref[pl.ds(_EMB * t, _EMB), :] = in_ref[:, pl.ds(_CH * t, _CH)]


def _tile_stream(tabT):
    return pl.pallas_call(
        _copy_body,
        grid=(_CPG,),
        in_specs=[pl.BlockSpec((_EMB, _CPW), lambda i: (0, i))],
        out_specs=pl.BlockSpec((_CPR, _CH), lambda i: (i, 0)),
        out_shape=jax.ShapeDtypeStruct((_NUSER * _EMB // _CH, _CH),
                                       jnp.float32),
    )(tabT)


def _sc_table_body(x1d, xlin, xb8, tailx,
                   x_o, bx_o,
                   xraw, xsh, xbsh, gx, gbx, tailxv, outx, outbx, sem):
    c = lax.axis_index("c")
    s = lax.axis_index("s")
    wid = s * _NC + c
    base = wid * _BPW
    pltpu.sync_copy(x1d.at[pl.ds(base, _BPW)], xraw)
    pltpu.sync_copy(tailx, tailxv)

    lane = jnp.arange(16, dtype=jnp.int32)

    def idx_body(t, carry):
        sl16 = pl.ds(t * 16, 16)
        x = xraw[sl16]
        xbsh[sl16] = x >> 3
        xc = jnp.minimum(x, _TAIL - 1)
        xbase = (xc >> 7) * 128 + ((xc >> 3) & 15)
        for d in range(_EMB):
            xsh[pl.ds(d * 512 + t * 16, 16)] = xbase + d * 16
        return carry
    lax.fori_loop(0, 32, idx_body, 0)

    copies = []
    for k in range(32):
        sl = pl.ds(k * _CH, _CH)
        copies.append(pltpu.async_copy(xlin.at[xsh.at[sl]], gx.at[sl], sem))
    for k in range(4):
        sl = pl.ds(k * _CH, _CH)
        copies.append(pltpu.async_copy(xb8.at[xbsh.at[sl]], gbx.at[sl], sem))
    for cp in copies:
        cp.wait()

    def sel_body(t, carry):
        sl16 = pl.ds(t * 16, 16)
        b = t * 16 + lane
        x = xraw[sl16]
        xsel = x & 7
        xtail = x >= _TAIL
        xtidx = jnp.maximum(x - _TAIL, 0)
        outbx[sl16] = plsc.load_gather(gbx, [b, xsel])
        for d in range(_EMB):
            p = d * 512 + b
            dsl = pl.ds(d * 512 + t * 16, 16)
            dvec = jnp.full((16,), d, jnp.int32)
            mx = plsc.load_gather(gx, [p, xsel])
            tx = plsc.load_gather(tailxv, [xtidx, dvec])
            outx[dsl] = jnp.where(xtail, tx, mx)
        return carry
    lax.fori_loop(0, 32, sel_body, 0)

    for d in range(_EMB):
        pltpu.sync_copy(outx.at[pl.ds(d * 512, 512)],
                        x_o.at[pl.ds(d * _BATCH + base, _BPW)])
    pltpu.sync_copy(outbx, bx_o.at[pl.ds(base, _BPW)])


def _sc_cat_body(c1d, catf, cbf, c_o, bc_o, craw, catv, cbv, outc, outbc):
    c = lax.axis_index("c")
    s = lax.axis_index("s")
    wid = s * _NC + c
    base = wid * _BPW
    pltpu.sync_copy(c1d.at[pl.ds(base, _BPW)], craw)
    pltpu.sync_copy(catf, catv)
    pltpu.sync_copy(cbf, cbv)

    def cat_body(t, carry):
        sl16 = pl.ds(t * 16, 16)
        cid = craw[sl16]
        outbc[sl16] = plsc.load_gather(cbv, [cid])
        for d in range(_EMB):
            outc[pl.ds(d * 512 + t * 16, 16)] = plsc.load_gather(
                catv, [d * _NCAT + cid])
        return carry
    lax.fori_loop(0, 32, cat_body, 0)

    for d in range(_EMB):
        pltpu.sync_copy(outc.at[pl.ds(d * 512, 512)],
                        c_o.at[pl.ds(d * _BATCH + base, _BPW)])
    pltpu.sync_copy(outbc, bc_o.at[pl.ds(base, _BPW)])


_sc_mesh = plsc.VectorSubcoreMesh(core_axis_name="c", subcore_axis_name="s")
_sc_params = pltpu.CompilerParams(use_tc_tiling_on_sc=False,
                                  needs_layout_passes=False)

_sc_table = functools.partial(
    pl.kernel,
    mesh=_sc_mesh,
    out_type=(
        jax.ShapeDtypeStruct((_BATCH * _EMB,), jnp.float32),
        jax.ShapeDtypeStruct((_BATCH,), jnp.float32),
    ),
    scratch_types=(
        [pltpu.VMEM((_BPW,), jnp.int32),
         pltpu.VMEM((_BPW * _EMB,), jnp.int32),
         pltpu.VMEM((_BPW,), jnp.int32),
         pltpu.VMEM((_BPW * _EMB, _EMB), jnp.float32),
         pltpu.VMEM((_BPW, _EMB), jnp.float32),
         pltpu.VMEM((_NUSER - _TAIL, _EMB), jnp.float32),
         pltpu.VMEM((_BPW * _EMB,), jnp.float32),
         pltpu.VMEM((_BPW,), jnp.float32),
         pltpu.SemaphoreType.DMA]),
    compiler_params=_sc_params,
)(_sc_table_body)

_sc_cat = functools.partial(
    pl.kernel,
    mesh=_sc_mesh,
    out_type=(
        jax.ShapeDtypeStruct((_BATCH * _EMB,), jnp.float32),
        jax.ShapeDtypeStruct((_BATCH,), jnp.float32),
    ),
    scratch_types=(
        [pltpu.VMEM((_BPW,), jnp.int32),
         pltpu.VMEM((_EMB * _NCAT,), jnp.float32),
         pltpu.VMEM((_NCAT,), jnp.float32),
         pltpu.VMEM((_BPW * _EMB,), jnp.float32),
         pltpu.VMEM((_BPW,), jnp.float32)]),
    compiler_params=_sc_params,
)(_sc_cat_body)


_BLK = 2048
_NBLK = _BATCH // _BLK


def _mlp_body(vis_ref, w1t_ref, b1_ref, w2t_ref, b2_ref, out_ref):
    h = jnp.maximum(
        lax.dot_general(vis_ref[...], w1t_ref[...],
                        (((1,), (1,)), ((), ())),
                        preferred_element_type=jnp.float32) + b1_ref[...], 0.0)
    v = lax.dot_general(h, w2t_ref[...], (((1,), (1,)), ((), ())),
                        preferred_element_type=jnp.float32) + b2_ref[...]
    out_ref[...] = v.T


def _combine_body(scal_ref, u_ref, i_ref, c_ref, v_ref,
                  bu_ref, bi_ref, bc_ref, out_ref):
    w = scal_ref[0, 0]
    vb = scal_ref[0, 1]
    mn = scal_ref[0, 2]
    i2 = (1.0 - w) * i_ref[...] + w * (v_ref[...] + c_ref[...])
    pred = jnp.sum(u_ref[...] * i2, axis=0)
    out_ref[0, 0, :] = (pred + bu_ref[0, 0, :] + bi_ref[0, 0, :]
                        + w * (vb + bc_ref[0, 0, :]) + mn)


def kernel(u_id, i_id, weight, visual_features, category_features,
           user_emb, user_bias, item_emb, item_bias,
           W1, b1, W2, b2, visual_bias, category_emb, category_bias, mean):
    u_id = u_id.astype(jnp.int32)
    i_id = i_id.astype(jnp.int32)
    cat = category_features.astype(jnp.int32)

    C1, bc = _sc_cat(cat, category_emb.T.reshape(_EMB * _NCAT),
                     category_bias.T.reshape(_NCAT))

    ulin = _tile_stream(user_emb.T).reshape(_NUSER, _EMB)
    U1, bu = _sc_table(u_id, ulin,
                       user_bias.T.reshape(_NUSER // _EMB, _EMB),
                       user_emb[_TAIL:])

    ilin = _tile_stream(item_emb.T).reshape(_NUSER, _EMB)
    I1, bi = _sc_table(i_id, ilin,
                       item_bias.T.reshape(_NUSER // _EMB, _EMB),
                       item_emb[_TAIL:])

    vt = pl.pallas_call(
        _mlp_body,
        grid=(_NBLK,),
        in_specs=[
            pl.BlockSpec((_BLK, _VIS), lambda i: (i, 0)),
            pl.BlockSpec((_HID, _VIS), lambda i: (0, 0)),
            pl.BlockSpec((1, _HID), lambda i: (0, 0)),
            pl.BlockSpec((_EMB, _HID), lambda i: (0, 0)),
            pl.BlockSpec((1, _EMB), lambda i: (0, 0)),
        ],
        out_specs=pl.BlockSpec((_EMB, _BLK), lambda i: (0, i)),
        out_shape=jax.ShapeDtypeStruct((_EMB, _BATCH), jnp.float32),
    )(visual_features, W1.T, b1.reshape(1, _HID), W2.T, b2.reshape(1, _EMB))

    scal = jnp.concatenate([weight, visual_bias, mean]).reshape(1, 3)

    out = pl.pallas_call(
        _combine_body,
        grid=(_NBLK,),
        in_specs=[
            pl.BlockSpec(memory_space=pltpu.SMEM),
            pl.BlockSpec((_EMB, _BLK), lambda i: (0, i)),
            pl.BlockSpec((_EMB, _BLK), lambda i: (0, i)),
            pl.BlockSpec((_EMB, _BLK), lambda i: (0, i)),
            pl.BlockSpec((_EMB, _BLK), lambda i: (0, i)),
            pl.BlockSpec((1, 1, _BLK), lambda i: (i, 0, 0)),
            pl.BlockSpec((1, 1, _BLK), lambda i: (i, 0, 0)),
            pl.BlockSpec((1, 1, _BLK), lambda i: (i, 0, 0)),
        ],
        out_specs=pl.BlockSpec((1, 1, _BLK), lambda i: (i, 0, 0)),
        out_shape=jax.ShapeDtypeStruct((_NBLK, 1, _BLK), jnp.float32),
    )(scal, U1.reshape(_EMB, _BATCH), I1.reshape(_EMB, _BATCH),
      C1.reshape(_EMB, _BATCH), vt,
      bu.reshape(_NBLK, 1, _BLK), bi.reshape(_NBLK, 1, _BLK),
      bc.reshape(_NBLK, 1, _BLK))

    return out.reshape(_BATCH)


# speedup vs baseline: 6.7673x; 1.1072x over previous
import functools

import jax
import jax.numpy as jnp
from jax import lax
from jax.experimental import pallas as pl
from jax.experimental.pallas import tpu as pltpu
from jax.experimental.pallas import tpu_sc as plsc

_BATCH = 16384
_VIS = 512
_EMB = 8
_HID = 16
_NUSER = 1000000
_NCAT = 368

_NC = 2
_NS = 16
_NW = _NC * _NS
_BPW = _BATCH // _NW
_CH = 128
_TAIL = (_NUSER // _CH) * _CH

_CPW = 16384
_CPG = (_NUSER + _CPW - 1) // _CPW
_CPR = _CPW // _CH * _EMB


def _copy_body(in_ref, out_ref):
    for t in range(_CPW // _CH):
        out_ref[pl.ds(_EMB * t, _EMB), :] = in_ref[:, pl.ds(_CH * t, _CH)]


def _tile_stream(tabT):
    return pl.pallas_call(
        _copy_body,
        grid=(_CPG,),
        in_specs=[pl.BlockSpec((_EMB, _CPW), lambda i: (0, i))],
        out_specs=pl.BlockSpec((_CPR, _CH), lambda i: (i, 0)),
        out_shape=jax.ShapeDtypeStruct((_NUSER * _EMB // _CH, _CH),
                                       jnp.float32),
    )(tabT)


_BSW = 16384
_BSG = (_NUSER + _BSW - 1) // _BSW


def _bias_body(in_ref, out_ref):
    out_ref[...] = in_ref[0, :]


def _bias_stream(biasT):
    return pl.pallas_call(
        _bias_body,
        grid=(_BSG,),
        in_specs=[pl.BlockSpec((1, _BSW), lambda i: (0, i))],
        out_specs=pl.BlockSpec((_BSW,), lambda i: (i,)),
        out_shape=jax.ShapeDtypeStruct((_NUSER,), jnp.float32),
    )(biasT)


def _sc_table_body(x1d, xlin, xb8, tailx,
                   x_o, bx_o,
                   xraw, xsh, xbsh, gx, gbx, tailxv, outx, outbx, sem):
    c = lax.axis_index("c")
    s = lax.axis_index("s")
    wid = s * _NC + c
    base = wid * _BPW
    pltpu.sync_copy(x1d.at[pl.ds(base, _BPW)], xraw)
    pltpu.sync_copy(tailx, tailxv)

    lane = jnp.arange(16, dtype=jnp.int32)

    def idx_body(t, carry):
        sl16 = pl.ds(t * 16, 16)
        x = xraw[sl16]
        xbsh[sl16] = x >> 3
        xc = jnp.minimum(x, _TAIL - 1)
        xbase = (xc >> 7) * 128 + ((xc >> 3) & 15)
        for d in range(_EMB):
            xsh[pl.ds(d * 512 + t * 16, 16)] = xbase + d * 16
        return carry
    lax.fori_loop(0, 32, idx_body, 0)

    copies = []
    for k in range(32):
        sl = pl.ds(k * _CH, _CH)
        copies.append(pltpu.async_copy(xlin.at[xsh.at[sl]], gx.at[sl], sem))
    for k in range(4):
        sl = pl.ds(k * _CH, _CH)
        copies.append(pltpu.async_copy(xb8.at[xbsh.at[sl]], gbx.at[sl], sem))
    for cp in copies:
        cp.wait()

    def sel_body(t, carry):
        sl16 = pl.ds(t * 16, 16)
        b = t * 16 + lane
        x = xraw[sl16]
        xsel = x & 7
        xtail = x >= _TAIL
        xtidx = jnp.maximum(x - _TAIL, 0)
        outbx[sl16] = plsc.load_gather(gbx, [b, xsel])
        for d in range(_EMB):
            p = d * 512 + b
            dsl = pl.ds(d * 512 + t * 16, 16)
            dvec = jnp.full((16,), d, jnp.int32)
            mx = plsc.load_gather(gx, [p, xsel])
            tx = plsc.load_gather(tailxv, [xtidx, dvec])
            outx[dsl] = jnp.where(xtail, tx, mx)
        return carry
    lax.fori_loop(0, 32, sel_body, 0)

    for d in range(_EMB):
        pltpu.sync_copy(outx.at[pl.ds(d * 512, 512)],
                        x_o.at[pl.ds(d * _BATCH + base, _BPW)])
    pltpu.sync_copy(outbx, bx_o.at[pl.ds(base, _BPW)])


def _sc_cat_body(c1d, catf, cbf, c_o, bc_o, craw, catv, cbv, outc, outbc):
    c = lax.axis_index("c")
    s = lax.axis_index("s")
    wid = s * _NC + c
    base = wid * _BPW
    pltpu.sync_copy(c1d.at[pl.ds(base, _BPW)], craw)
    pltpu.sync_copy(catf, catv)
    pltpu.sync_copy(cbf, cbv)

    def cat_body(t, carry):
        sl16 = pl.ds(t * 16, 16)
        cid = craw[sl16]
        outbc[sl16] = plsc.load_gather(cbv, [cid])
        for d in range(_EMB):
            outc[pl.ds(d * 512 + t * 16, 16)] = plsc.load_gather(
                catv, [d * _NCAT + cid])
        return carry
    lax.fori_loop(0, 32, cat_body, 0)

    for d in range(_EMB):
        pltpu.sync_copy(outc.at[pl.ds(d * 512, 512)],
                        c_o.at[pl.ds(d * _BATCH + base, _BPW)])
    pltpu.sync_copy(outbc, bc_o.at[pl.ds(base, _BPW)])


_sc_mesh = plsc.VectorSubcoreMesh(core_axis_name="c", subcore_axis_name="s")
_sc_params = pltpu.CompilerParams(use_tc_tiling_on_sc=False,
                                  needs_layout_passes=False)

_sc_table = functools.partial(
    pl.kernel,
    mesh=_sc_mesh,
    out_type=(
        jax.ShapeDtypeStruct((_BATCH * _EMB,), jnp.float32),
        jax.ShapeDtypeStruct((_BATCH,), jnp.float32),
    ),
    scratch_types=(
        [pltpu.VMEM((_BPW,), jnp.int32),
         pltpu.VMEM((_BPW * _EMB,), jnp.int32),
         pltpu.VMEM((_BPW,), jnp.int32),
         pltpu.VMEM((_BPW * _EMB, _EMB), jnp.float32),
         pltpu.VMEM((_BPW, _EMB), jnp.float32),
         pltpu.VMEM((_NUSER - _TAIL, _EMB), jnp.float32),
         pltpu.VMEM((_BPW * _EMB,), jnp.float32),
         pltpu.VMEM((_BPW,), jnp.float32),
         pltpu.SemaphoreType.DMA]),
    compiler_params=_sc_params,
)(_sc_table_body)

_sc_cat = functools.partial(
    pl.kernel,
    mesh=_sc_mesh,
    out_type=(
        jax.ShapeDtypeStruct((_BATCH * _EMB,), jnp.float32),
        jax.ShapeDtypeStruct((_BATCH,), jnp.float32),
    ),
    scratch_types=(
        [pltpu.VMEM((_BPW,), jnp.int32),
         pltpu.VMEM((_EMB * _NCAT,), jnp.float32),
         pltpu.VMEM((_NCAT,), jnp.float32),
         pltpu.VMEM((_BPW * _EMB,), jnp.float32),
         pltpu.VMEM((_BPW,), jnp.float32)]),
    compiler_params=_sc_params,
)(_sc_cat_body)


_BLK = 2048
_NBLK = _BATCH // _BLK


def _mlp_body(vis_ref, w1t_ref, b1_ref, w2t_ref, b2_ref, out_ref):
    h = jnp.maximum(
        lax.dot_general(vis_ref[...], w1t_ref[...],
                        (((1,), (1,)), ((), ())),
                        preferred_element_type=jnp.float32) + b1_ref[...], 0.0)
    v = lax.dot_general(h, w2t_ref[...], (((1,), (1,)), ((), ())),
                        preferred_element_type=jnp.float32) + b2_ref[...]
    out_ref[...] = v.T


def _combine_body(scal_ref, u_ref, i_ref, c_ref, v_ref,
                  bu_ref, bi_ref, bc_ref, out_ref):
    w = scal_ref[0, 0]
    vb = scal_ref[0, 1]
    mn = scal_ref[0, 2]
    i2 = (1.0 - w) * i_ref[...] + w * (v_ref[...] + c_ref[...])
    pred = jnp.sum(u_ref[...] * i2, axis=0)
    out_ref[0, 0, :] = (pred + bu_ref[0, 0, :] + bi_ref[0, 0, :]
                        + w * (vb + bc_ref[0, 0, :]) + mn)


def kernel(u_id, i_id, weight, visual_features, category_features,
           user_emb, user_bias, item_emb, item_bias,
           W1, b1, W2, b2, visual_bias, category_emb, category_bias, mean):
    u_id = u_id.astype(jnp.int32)
    i_id = i_id.astype(jnp.int32)
    cat = category_features.astype(jnp.int32)

    C1, bc = _sc_cat(cat, category_emb.T.reshape(_EMB * _NCAT),
                     category_bias.T.reshape(_NCAT))

    ulin = _tile_stream(user_emb.T).reshape(_NUSER, _EMB)
    U1, bu = _sc_table(u_id, ulin,
                       _bias_stream(user_bias.T).reshape(_NUSER // _EMB, _EMB),
                       user_emb[_TAIL:])

    ilin = _tile_stream(item_emb.T).reshape(_NUSER, _EMB)
    I1, bi = _sc_table(i_id, ilin,
                       _bias_stream(item_bias.T).reshape(_NUSER // _EMB, _EMB),
                       item_emb[_TAIL:])

    vt = pl.pallas_call(
        _mlp_body,
        grid=(_NBLK,),
        in_specs=[
            pl.BlockSpec((_BLK, _VIS), lambda i: (i, 0)),
            pl.BlockSpec((_HID, _VIS), lambda i: (0, 0)),
            pl.BlockSpec((1, _HID), lambda i: (0, 0)),
            pl.BlockSpec((_EMB, _HID), lambda i: (0, 0)),
            pl.BlockSpec((1, _EMB), lambda i: (0, 0)),
        ],
        out_specs=pl.BlockSpec((_EMB, _BLK), lambda i: (0, i)),
        out_shape=jax.ShapeDtypeStruct((_EMB, _BATCH), jnp.float32),
    )(visual_features, W1.T, b1.reshape(1, _HID), W2.T, b2.reshape(1, _EMB))

    scal = jnp.concatenate([weight, visual_bias, mean]).reshape(1, 3)

    out = pl.pallas_call(
        _combine_body,
        grid=(_NBLK,),
        in_specs=[
            pl.BlockSpec(memory_space=pltpu.SMEM),
            pl.BlockSpec((_EMB, _BLK), lambda i: (0, i)),
            pl.BlockSpec((_EMB, _BLK), lambda i: (0, i)),
            pl.BlockSpec((_EMB, _BLK), lambda i: (0, i)),
            pl.BlockSpec((_EMB, _BLK), lambda i: (0, i)),
            pl.BlockSpec((1, 1, _BLK), lambda i: (i, 0, 0)),
            pl.BlockSpec((1, 1, _BLK), lambda i: (i, 0, 0)),
            pl.BlockSpec((1, 1, _BLK), lambda i: (i, 0, 0)),
        ],
        out_specs=pl.BlockSpec((1, 1, _BLK), lambda i: (i, 0, 0)),
        out_shape=jax.ShapeDtypeStruct((_NBLK, 1, _BLK), jnp.float32),
    )(scal, U1.reshape(_EMB, _BATCH), I1.reshape(_EMB, _BATCH),
      C1.reshape(_EMB, _BATCH), vt,
      bu.reshape(_NBLK, 1, _BLK), bi.reshape(_NBLK, 1, _BLK),
      bc.reshape(_NBLK, 1, _BLK))

    return out.reshape(_BATCH)


# speedup vs baseline: 9.5617x; 1.4129x over previous
import functools

import jax
import jax.numpy as jnp
from jax import lax
from jax.experimental import pallas as pl
from jax.experimental.pallas import tpu as pltpu
from jax.experimental.pallas import tpu_sc as plsc

_BATCH = 16384
_VIS = 512
_EMB = 8
_HID = 16
_NUSER = 1000000
_NCAT = 368

_NC = 2
_NS = 16
_NW = _NC * _NS
_BPW = _BATCH // _NW
_CH = 128
_TAIL = (_NUSER // _CH) * _CH

_CPW = 16384
_CPG = (_NUSER + _CPW - 1) // _CPW
_CPR = _CPW // _CH * _EMB


def _copy_body(in_ref, bias_ref, out_ref, bout_ref):
    for t in range(_CPW // _CH):
        out_ref[pl.ds(_EMB * t, _EMB), :] = in_ref[:, pl.ds(_CH * t, _CH)]
    bout_ref[...] = bias_ref[0, :]


def _tile_stream(tabT, biasT):
    return pl.pallas_call(
        _copy_body,
        grid=(_CPG,),
        in_specs=[pl.BlockSpec((_EMB, _CPW), lambda i: (0, i)),
                  pl.BlockSpec((1, _CPW), lambda i: (0, i))],
        out_specs=[pl.BlockSpec((_CPR, _CH), lambda i: (i, 0)),
                   pl.BlockSpec((_CPW,), lambda i: (i,))],
        out_shape=[jax.ShapeDtypeStruct((_NUSER * _EMB // _CH, _CH),
                                        jnp.float32),
                   jax.ShapeDtypeStruct((_NUSER,), jnp.float32)],
    )(tabT, biasT)


_BSW = 16384
_BSG = (_NUSER + _BSW - 1) // _BSW


def _bias_body(in_ref, out_ref):
    out_ref[...] = in_ref[0, :]


def _bias_stream(biasT):
    return pl.pallas_call(
        _bias_body,
        grid=(_BSG,),
        in_specs=[pl.BlockSpec((1, _BSW), lambda i: (0, i))],
        out_specs=pl.BlockSpec((_BSW,), lambda i: (i,)),
        out_shape=jax.ShapeDtypeStruct((_NUSER,), jnp.float32),
    )(biasT)


def _sc_table_body(x1d, xlin, xb8, tailx,
                   x_o, bx_o,
                   xraw, xsh, xbsh, gx, gbx, tailxv, outx, outbx, sem):
    c = lax.axis_index("c")
    s = lax.axis_index("s")
    wid = s * _NC + c
    base = wid * _BPW
    pltpu.sync_copy(x1d.at[pl.ds(base, _BPW)], xraw)
    pltpu.sync_copy(tailx, tailxv)

    lane = jnp.arange(16, dtype=jnp.int32)

    def idx_body(t, carry):
        sl16 = pl.ds(t * 16, 16)
        x = xraw[sl16]
        xbsh[sl16] = x >> 3
        xc = jnp.minimum(x, _TAIL - 1)
        xbase = (xc >> 7) * 128 + ((xc >> 3) & 15)
        for d in range(_EMB):
            xsh[pl.ds(d * 512 + t * 16, 16)] = xbase + d * 16
        return carry
    lax.fori_loop(0, 32, idx_body, 0)

    copies = []
    for k in range(32):
        sl = pl.ds(k * _CH, _CH)
        copies.append(pltpu.async_copy(xlin.at[xsh.at[sl]], gx.at[sl], sem))
    for k in range(4):
        sl = pl.ds(k * _CH, _CH)
        copies.append(pltpu.async_copy(xb8.at[xbsh.at[sl]], gbx.at[sl], sem))
    for cp in copies:
        cp.wait()

    def sel_body(t, carry):
        sl16 = pl.ds(t * 16, 16)
        b = t * 16 + lane
        x = xraw[sl16]
        xsel = x & 7
        xtail = x >= _TAIL
        xtidx = jnp.maximum(x - _TAIL, 0)
        outbx[sl16] = plsc.load_gather(gbx, [b, xsel])
        for d in range(_EMB):
            p = d * 512 + b
            dsl = pl.ds(d * 512 + t * 16, 16)
            dvec = jnp.full((16,), d, jnp.int32)
            mx = plsc.load_gather(gx, [p, xsel])
            tx = plsc.load_gather(tailxv, [xtidx, dvec])
            outx[dsl] = jnp.where(xtail, tx, mx)
        return carry
    lax.fori_loop(0, 32, sel_body, 0)

    for d in range(_EMB):
        pltpu.sync_copy(outx.at[pl.ds(d * 512, 512)],
                        x_o.at[pl.ds(d * _BATCH + base, _BPW)])
    pltpu.sync_copy(outbx, bx_o.at[pl.ds(base, _BPW)])


def _sc_cat_body(c1d, catf, cbf, c_o, bc_o, craw, catv, cbv, outc, outbc):
    c = lax.axis_index("c")
    s = lax.axis_index("s")
    wid = s * _NC + c
    base = wid * _BPW
    pltpu.sync_copy(c1d.at[pl.ds(base, _BPW)], craw)
    pltpu.sync_copy(catf, catv)
    pltpu.sync_copy(cbf, cbv)

    def cat_body(t, carry):
        sl16 = pl.ds(t * 16, 16)
        cid = craw[sl16]
        outbc[sl16] = plsc.load_gather(cbv, [cid])
        for d in range(_EMB):
            outc[pl.ds(d * 512 + t * 16, 16)] = plsc.load_gather(
                catv, [d * _NCAT + cid])
        return carry
    lax.fori_loop(0, 32, cat_body, 0)

    for d in range(_EMB):
        pltpu.sync_copy(outc.at[pl.ds(d * 512, 512)],
                        c_o.at[pl.ds(d * _BATCH + base, _BPW)])
    pltpu.sync_copy(outbc, bc_o.at[pl.ds(base, _BPW)])


_sc_mesh = plsc.VectorSubcoreMesh(core_axis_name="c", subcore_axis_name="s")
_sc_params = pltpu.CompilerParams(use_tc_tiling_on_sc=False,
                                  needs_layout_passes=False)

_sc_table = functools.partial(
    pl.kernel,
    mesh=_sc_mesh,
    out_type=(
        jax.ShapeDtypeStruct((_BATCH * _EMB,), jnp.float32),
        jax.ShapeDtypeStruct((_BATCH,), jnp.float32),
    ),
    scratch_types=(
        [pltpu.VMEM((_BPW,), jnp.int32),
         pltpu.VMEM((_BPW * _EMB,), jnp.int32),
         pltpu.VMEM((_BPW,), jnp.int32),
         pltpu.VMEM((_BPW * _EMB, _EMB), jnp.float32),
         pltpu.VMEM((_BPW, _EMB), jnp.float32),
         pltpu.VMEM((_NUSER - _TAIL, _EMB), jnp.float32),
         pltpu.VMEM((_BPW * _EMB,), jnp.float32),
         pltpu.VMEM((_BPW,), jnp.float32),
         pltpu.SemaphoreType.DMA]),
    compiler_params=_sc_params,
)(_sc_table_body)

_sc_cat = functools.partial(
    pl.kernel,
    mesh=_sc_mesh,
    out_type=(
        jax.ShapeDtypeStruct((_BATCH * _EMB,), jnp.float32),
        jax.ShapeDtypeStruct((_BATCH,), jnp.float32),
    ),
    scratch_types=(
        [pltpu.VMEM((_BPW,), jnp.int32),
         pltpu.VMEM((_EMB * _NCAT,), jnp.float32),
         pltpu.VMEM((_NCAT,), jnp.float32),
         pltpu.VMEM((_BPW * _EMB,), jnp.float32),
         pltpu.VMEM((_BPW,), jnp.float32)]),
    compiler_params=_sc_params,
)(_sc_cat_body)


_BLK = 2048
_NBLK = _BATCH // _BLK


def _mlp_body(vis_ref, w1t_ref, b1_ref, w2t_ref, b2_ref, out_ref):
    h = jnp.maximum(
        lax.dot_general(vis_ref[...], w1t_ref[...],
                        (((1,), (1,)), ((), ())),
                        preferred_element_type=jnp.float32) + b1_ref[...], 0.0)
    v = lax.dot_general(h, w2t_ref[...], (((1,), (1,)), ((), ())),
                        preferred_element_type=jnp.float32) + b2_ref[...]
    out_ref[...] = v.T


def _combine_body(scal_ref, u_ref, i_ref, c_ref, v_ref,
                  bu_ref, bi_ref, bc_ref, out_ref):
    w = scal_ref[0, 0]
    vb = scal_ref[0, 1]
    mn = scal_ref[0, 2]
    i2 = (1.0 - w) * i_ref[...] + w * (v_ref[...] + c_ref[...])
    pred = jnp.sum(u_ref[...] * i2, axis=0)
    out_ref[0, 0, :] = (pred + bu_ref[0, 0, :] + bi_ref[0, 0, :]
                        + w * (vb + bc_ref[0, 0, :]) + mn)


def kernel(u_id, i_id, weight, visual_features, category_features,
           user_emb, user_bias, item_emb, item_bias,
           W1, b1, W2, b2, visual_bias, category_emb, category_bias, mean):
    u_id = u_id.astype(jnp.int32)
    i_id = i_id.astype(jnp.int32)
    cat = category_features.astype(jnp.int32)

    C1, bc = _sc_cat(cat, category_emb.T.reshape(_EMB * _NCAT),
                     category_bias.T.reshape(_NCAT))

    ustream, ub1 = _tile_stream(user_emb.T, user_bias.T)
    U1, bu = _sc_table(u_id, ustream.reshape(_NUSER, _EMB),
                       ub1.reshape(_NUSER // _EMB, _EMB),
                       user_emb[_TAIL:])

    istream, ib1 = _tile_stream(item_emb.T, item_bias.T)
    I1, bi = _sc_table(i_id, istream.reshape(_NUSER, _EMB),
                       ib1.reshape(_NUSER // _EMB, _EMB),
                       item_emb[_TAIL:])

    vt = pl.pallas_call(
        _mlp_body,
        grid=(_NBLK,),
        in_specs=[
            pl.BlockSpec((_BLK, _VIS), lambda i: (i, 0)),
            pl.BlockSpec((_HID, _VIS), lambda i: (0, 0)),
            pl.BlockSpec((1, _HID), lambda i: (0, 0)),
            pl.BlockSpec((_EMB, _HID), lambda i: (0, 0)),
            pl.BlockSpec((1, _EMB), lambda i: (0, 0)),
        ],
        out_specs=pl.BlockSpec((_EMB, _BLK), lambda i: (0, i)),
        out_shape=jax.ShapeDtypeStruct((_EMB, _BATCH), jnp.float32),
    )(visual_features, W1.T, b1.reshape(1, _HID), W2.T, b2.reshape(1, _EMB))

    scal = jnp.concatenate([weight, visual_bias, mean]).reshape(1, 3)

    out = pl.pallas_call(
        _combine_body,
        grid=(_NBLK,),
        in_specs=[
            pl.BlockSpec(memory_space=pltpu.SMEM),
            pl.BlockSpec((_EMB, _BLK), lambda i: (0, i)),
            pl.BlockSpec((_EMB, _BLK), lambda i: (0, i)),
            pl.BlockSpec((_EMB, _BLK), lambda i: (0, i)),
            pl.BlockSpec((_EMB, _BLK), lambda i: (0, i)),
            pl.BlockSpec((1, 1, _BLK), lambda i: (i, 0, 0)),
            pl.BlockSpec((1, 1, _BLK), lambda i: (i, 0, 0)),
            pl.BlockSpec((1, 1, _BLK), lambda i: (i, 0, 0)),
        ],
        out_specs=pl.BlockSpec((1, 1, _BLK), lambda i: (i, 0, 0)),
        out_shape=jax.ShapeDtypeStruct((_NBLK, 1, _BLK), jnp.float32),
    )(scal, U1.reshape(_EMB, _BATCH), I1.reshape(_EMB, _BATCH),
      C1.reshape(_EMB, _BATCH), vt,
      bu.reshape(_NBLK, 1, _BLK), bi.reshape(_NBLK, 1, _BLK),
      bc.reshape(_NBLK, 1, _BLK))

    return out.reshape(_BATCH)
